# Initial kernel scaffold; baseline (speedup 1.0000x reference)
import jax
import jax.numpy as jnp
from jax.experimental import pallas as pl


def kernel(input, W_enc, b_enc, W1, b1, W2, b2, W3, b3):
    raise NotImplementedError("write your pallas kernel here")


# speedup vs baseline: 9.2707x; 9.2707x over previous
import jax
import jax.numpy as jnp
from jax.experimental import pallas as pl

_GRID = 0.01


def _pool_kernel(x_ref, w_ref, b_ref, o_ref):
    x = x_ref[0]
    cf = jnp.floor(x / _GRID).astype(jnp.int32).astype(jnp.float32) * _GRID
    h = jax.lax.dot_general(
        cf, w_ref[:], (((1,), (0,)), ((), ())),
        preferred_element_type=jnp.float32,
    )
    h = jnp.maximum(h + b_ref[:], 0.0)
    o_ref[0, 0, :] = jnp.max(h, axis=0)


def _mlp_kernel(p_ref, w1_ref, b1_ref, w2_ref, b2_ref, w3_ref, b3_ref, o_ref):
    x = jnp.maximum(
        jnp.dot(p_ref[:], w1_ref[:], preferred_element_type=jnp.float32)
        + b1_ref[:], 0.0)
    x = jnp.maximum(
        jnp.dot(x, w2_ref[:], preferred_element_type=jnp.float32)
        + b2_ref[:], 0.0)
    o_ref[:] = (
        jnp.dot(x, w3_ref[:], preferred_element_type=jnp.float32) + b3_ref[:])


def kernel(input, W_enc, b_enc, W1, b1, W2, b2, W3, b3):
    if input.shape[-1] != 3:
        input = jnp.transpose(input, (0, 2, 1))
    B, N = input.shape[0], input.shape[1]
    F = W_enc.shape[1]

    w_coords = W_enc[1:4]
    bias0 = (b_enc + W_enc[0]).reshape(1, F)

    pooled = pl.pallas_call(
        _pool_kernel,
        grid=(B,),
        in_specs=[
            pl.BlockSpec((1, N, 3), lambda b: (b, 0, 0)),
            pl.BlockSpec((3, F), lambda b: (0, 0)),
            pl.BlockSpec((1, F), lambda b: (0, 0)),
        ],
        out_specs=pl.BlockSpec((1, 1, F), lambda b: (b, 0, 0)),
        out_shape=jax.ShapeDtypeStruct((B, 1, F), jnp.float32),
    )(input, w_coords, bias0)
    pooled = pooled.reshape(B, F)

    H1, H2, P = W1.shape[1], W2.shape[1], W3.shape[1]
    PP = 128
    W3p = jnp.pad(W3, ((0, 0), (0, PP - P)))
    b3p = jnp.pad(b3, (0, PP - P)).reshape(1, PP)

    pose = pl.pallas_call(
        _mlp_kernel,
        in_specs=[
            pl.BlockSpec((B, F), lambda: (0, 0)),
            pl.BlockSpec((F, H1), lambda: (0, 0)),
            pl.BlockSpec((1, H1), lambda: (0, 0)),
            pl.BlockSpec((H1, H2), lambda: (0, 0)),
            pl.BlockSpec((1, H2), lambda: (0, 0)),
            pl.BlockSpec((H2, PP), lambda: (0, 0)),
            pl.BlockSpec((1, PP), lambda: (0, 0)),
        ],
        out_specs=pl.BlockSpec((B, PP), lambda: (0, 0)),
        out_shape=jax.ShapeDtypeStruct((B, PP), jnp.float32),
    )(pooled, W1, b1.reshape(1, H1), W2, b2.reshape(1, H2), W3p, b3p)

    return pose[:, :P]


# speedup vs baseline: 12.0188x; 1.2964x over previous
import jax
import jax.numpy as jnp
from jax.experimental import pallas as pl
from jax.experimental.pallas import tpu as pltpu

_GRID = 0.01


def _fused_kernel(x_ref, w_ref, bias_ref, w1_ref, b1_ref, w2_ref, b2_ref,
                  w3_ref, b3_ref, o_ref, acc_ref):
    b = pl.program_id(0)
    nb = pl.num_programs(0)
    xt = x_ref[0]
    cf = jnp.floor(xt / _GRID) * _GRID
    h = jax.lax.dot_general(
        cf, w_ref[:], (((0,), (0,)), ((), ())),
        preferred_element_type=jnp.float32,
    )
    acc_ref[pl.ds(b, 1), :] = jnp.max(h, axis=0, keepdims=True)

    @pl.when(b == nb - 1)
    def _mlp():
        pooled = jnp.maximum(acc_ref[:, :] + bias_ref[:], 0.0)
        x1 = jnp.maximum(
            jnp.dot(pooled, w1_ref[:], preferred_element_type=jnp.float32)
            + b1_ref[:], 0.0)
        x2 = jnp.maximum(
            jnp.dot(x1, w2_ref[:], preferred_element_type=jnp.float32)
            + b2_ref[:], 0.0)
        o_ref[:] = (
            jnp.dot(x2, w3_ref[:], preferred_element_type=jnp.float32)
            + b3_ref[:])


def kernel(input, W_enc, b_enc, W1, b1, W2, b2, W3, b3):
    if input.shape[-1] != 3:
        input = jnp.transpose(input, (0, 2, 1))
    B, N = input.shape[0], input.shape[1]
    F = W_enc.shape[1]
    H1, H2, P = W1.shape[1], W2.shape[1], W3.shape[1]
    PP = 128

    xt = jnp.transpose(input, (0, 2, 1))
    w_coords = W_enc[1:4]
    bias0 = (b_enc + W_enc[0]).reshape(1, F)
    W3p = jnp.pad(W3, ((0, 0), (0, PP - P)))
    b3p = jnp.pad(b3, (0, PP - P)).reshape(1, PP)

    pose = pl.pallas_call(
        _fused_kernel,
        grid=(B,),
        in_specs=[
            pl.BlockSpec((1, 3, N), lambda b: (b, 0, 0)),
            pl.BlockSpec((3, F), lambda b: (0, 0)),
            pl.BlockSpec((1, F), lambda b: (0, 0)),
            pl.BlockSpec((F, H1), lambda b: (0, 0)),
            pl.BlockSpec((1, H1), lambda b: (0, 0)),
            pl.BlockSpec((H1, H2), lambda b: (0, 0)),
            pl.BlockSpec((1, H2), lambda b: (0, 0)),
            pl.BlockSpec((H2, PP), lambda b: (0, 0)),
            pl.BlockSpec((1, PP), lambda b: (0, 0)),
        ],
        out_specs=pl.BlockSpec((B, PP), lambda b: (0, 0)),
        out_shape=jax.ShapeDtypeStruct((B, PP), jnp.float32),
        scratch_shapes=[pltpu.VMEM((B, F), jnp.float32)],
    )(xt, w_coords, bias0, W1, b1.reshape(1, H1), W2, b2.reshape(1, H2),
      W3p, b3p)

    return pose[:, :P]
